# baseline (device time: 47909 ns/iter reference)
import jax
import jax.numpy as jnp
from jax import lax
from jax.experimental import pallas as pl
from jax.experimental.pallas import tpu as pltpu

N_DEV = 4
M_BLK = 2048 // N_DEV
NQ = 4


def kernel(x, w_mat):
    m, k_per = x.shape
    _, n = w_mat.shape
    qw = n // NQ

    def body(x_ref, w_ref, out_ref,
             xv, wv, oq,
             s1A, r1A, s1B, r1B, s2A, r2A, s2B, r2B,
             in_sems, out_sems,
             send1_sems, recv1_sems, send2_sems, recv2_sems):
        p = lax.axis_index("i")
        p1 = jnp.bitwise_xor(p, 1)
        p2 = 3 - p
        diag = 3 - p1

        xblocks = [diag, p1, p2, p]
        xdma = []
        for i, c in enumerate(xblocks):
            d = pltpu.make_async_copy(
                x_ref.at[pl.ds(c * M_BLK, M_BLK), :], xv.at[i],
                in_sems.at[i],
            )
            xdma.append(d)
        wdma = []
        for q in range(NQ):
            d = pltpu.make_async_copy(
                w_ref.at[:, q * qw:(q + 1) * qw], wv.at[q],
                in_sems.at[NQ + q],
            )
            wdma.append(d)
        xdma[0].start()
        for d in wdma:
            d.start()
        for d in xdma[1:]:
            d.start()

        barrier_sem = pltpu.get_barrier_semaphore()
        for nbr in [p1, p2]:
            pl.semaphore_signal(
                barrier_sem, inc=1,
                device_id=(nbr,), device_id_type=pl.DeviceIdType.MESH,
            )
        pl.semaphore_wait(barrier_sem, 2)

        xbf = [None] * 4
        wbf = [None] * 4

        def need_x(i):
            if xbf[i] is None:
                xdma[i].wait()
                xbf[i] = xv[i].astype(jnp.bfloat16)
            return xbf[i]

        def need_w(q):
            if wbf[q] is None:
                wdma[q].wait()
                wbf[q] = wv[q].astype(jnp.bfloat16)
            return wbf[q]

        def pq(i, q):
            return jnp.dot(need_x(i), need_w(q),
                           preferred_element_type=jnp.float32)

        def mk1(src, dst, i, dev):
            return pltpu.make_async_remote_copy(
                src_ref=src, dst_ref=dst,
                send_sem=send1_sems.at[i], recv_sem=recv1_sems.at[i],
                device_id=(dev,), device_id_type=pl.DeviceIdType.MESH,
            )

        def mk2(src, dst, i, dev):
            return pltpu.make_async_remote_copy(
                src_ref=src, dst_ref=dst,
                send_sem=send2_sems.at[i], recv_sem=recv2_sems.at[i],
                device_id=(dev,), device_id_type=pl.DeviceIdType.MESH,
            )

        srcsA = [(0, 0, 0, 0), (0, 1, 0, 1), (1, 0, 1, 0), (1, 1, 1, 1)]
        srcsB = [(0, 0, 0, 2), (0, 1, 0, 3), (1, 0, 2, 2), (1, 1, 2, 3)]
        dA, dB = [], []
        for i in range(4):
            blkA, subA, roleA, qA = srcsA[i]
            s1A[blkA, subA] = pq(roleA, qA).astype(jnp.bfloat16)
            d = mk1(s1A.at[blkA, subA], r1A.at[blkA, subA], i, p1)
            d.start()
            dA.append(d)
            blkB, subB, roleB, qB = srcsB[i]
            s1B[blkB, subB] = pq(roleB, qB).astype(jnp.bfloat16)
            d = mk1(s1B.at[blkB, subB], r1B.at[blkB, subB], 4 + i, p2)
            d.start()
            dB.append(d)

        pA_fwd = [pq(2, 0), pq(2, 1)]
        pB_fwd = [pq(1, 2), pq(1, 3)]

        r2_descs = []
        for i in range(2):
            dA[i].wait_recv()
            s2A[i] = (r1A[0, i].astype(jnp.float32) + pA_fwd[i]).astype(jnp.bfloat16)
            d = mk2(s2A.at[i], r2A.at[i], i, p2)
            d.start()
            r2_descs.append(d)
            dB[i].wait_recv()
            s2B[i] = (r1B[0, i].astype(jnp.float32) + pB_fwd[i]).astype(jnp.bfloat16)
            d = mk2(s2B.at[i], r2B.at[i], 2 + i, p1)
            d.start()
            r2_descs.append(d)

        pA_own = [pq(3, 0), pq(3, 1)]
        pB_own = [pq(3, 2), pq(3, 3)]
        dA[2].wait_recv()
        accA = [pA_own[0] + r1A[1, 0].astype(jnp.float32)]
        dA[3].wait_recv()
        accA.append(pA_own[1] + r1A[1, 1].astype(jnp.float32))
        dB[2].wait_recv()
        accB = [pB_own[0] + r1B[1, 0].astype(jnp.float32)]
        dB[3].wait_recv()
        accB.append(pB_own[1] + r1B[1, 1].astype(jnp.float32))

        odma = []
        for i in range(2):
            r2_descs[2 * i].wait_recv()
            oq[i] = accA[i] + r2A[i].astype(jnp.float32)
            d = pltpu.make_async_copy(
                oq.at[i], out_ref.at[:, i * qw:(i + 1) * qw],
                out_sems.at[i],
            )
            d.start()
            odma.append(d)
            r2_descs[2 * i + 1].wait_recv()
            oq[2 + i] = accB[i] + r2B[i].astype(jnp.float32)
            d = pltpu.make_async_copy(
                oq.at[2 + i], out_ref.at[:, (2 + i) * qw:(3 + i) * qw],
                out_sems.at[2 + i],
            )
            d.start()
            odma.append(d)

        for d in odma:
            d.wait()
        for d in dA + dB + r2_descs:
            d.wait_send()

    blk_q = (M_BLK, qw)
    return pl.pallas_call(
        body,
        out_shape=jax.ShapeDtypeStruct((M_BLK, n), jnp.float32),
        in_specs=[
            pl.BlockSpec(memory_space=pltpu.MemorySpace.HBM),
            pl.BlockSpec(memory_space=pltpu.MemorySpace.HBM),
        ],
        out_specs=pl.BlockSpec(memory_space=pltpu.MemorySpace.HBM),
        scratch_shapes=[
            pltpu.VMEM((4, M_BLK, k_per), jnp.float32),
            pltpu.VMEM((NQ, k_per, qw), jnp.float32),
            pltpu.VMEM((NQ, M_BLK, qw), jnp.float32),
            pltpu.VMEM((2, 2) + blk_q, jnp.bfloat16),
            pltpu.VMEM((2, 2) + blk_q, jnp.bfloat16),
            pltpu.VMEM((2, 2) + blk_q, jnp.bfloat16),
            pltpu.VMEM((2, 2) + blk_q, jnp.bfloat16),
            pltpu.VMEM((2,) + blk_q, jnp.bfloat16),
            pltpu.VMEM((2,) + blk_q, jnp.bfloat16),
            pltpu.VMEM((2,) + blk_q, jnp.bfloat16),
            pltpu.VMEM((2,) + blk_q, jnp.bfloat16),
            pltpu.SemaphoreType.DMA((8,)),
            pltpu.SemaphoreType.DMA((4,)),
            pltpu.SemaphoreType.DMA((8,)),
            pltpu.SemaphoreType.DMA((8,)),
            pltpu.SemaphoreType.DMA((4,)),
            pltpu.SemaphoreType.DMA((4,)),
        ],
        compiler_params=pltpu.CompilerParams(collective_id=0),
    )(x, w_mat)


# device time: 46414 ns/iter; 1.0322x vs baseline; 1.0322x over previous
import jax
import jax.numpy as jnp
from jax import lax
from jax.experimental import pallas as pl
from jax.experimental.pallas import tpu as pltpu

N_DEV = 4
M_BLK = 2048 // N_DEV
S = 2


def kernel(x, w_mat):
    m, k_per = x.shape
    _, n = w_mat.shape
    nh = n // 2
    swid = nh // S

    def body(x_ref, w_ref, out_ref,
             xbf_ref, commR_ref, commL_ref,
             sendR_sems, recvR_sems, sendL_sems, recvL_sems):
        p = lax.axis_index("i")
        left = lax.rem(p + N_DEV - 1, N_DEV)
        right = lax.rem(p + 1, N_DEV)

        barrier_sem = pltpu.get_barrier_semaphore()
        for nbr in [left, right]:
            pl.semaphore_signal(
                barrier_sem, inc=1,
                device_id=(nbr,), device_id_type=pl.DeviceIdType.MESH,
            )
        pl.semaphore_wait(barrier_sem, 2)

        xbf_ref[:, :] = x_ref[:, :].astype(jnp.bfloat16)
        w_bf = w_ref[:, :].astype(jnp.bfloat16)

        def pq(c, q):
            xc = xbf_ref[pl.ds(c * M_BLK, M_BLK), :]
            wc = w_bf[:, q * swid:(q + 1) * swid]
            return jnp.dot(xc, wc, preferred_element_type=jnp.float32)

        def blkR(h):
            return lax.rem(p + 2 * N_DEV - h - 2, N_DEV)

        def blkL(h):
            return lax.rem(p + h + 2, N_DEV)

        def mk(comm_ref, sems_s, sems_r, h, j, dst):
            return pltpu.make_async_remote_copy(
                src_ref=comm_ref.at[h, j],
                dst_ref=comm_ref.at[h + 1, j],
                send_sem=sems_s.at[h, j],
                recv_sem=sems_r.at[h, j],
                device_id=(dst,),
                device_id_type=pl.DeviceIdType.MESH,
            )

        all_rdmas = []

        bR0 = lax.rem(p + N_DEV - 1, N_DEV)
        bL0 = lax.rem(p + 1, N_DEV)
        curR, curL = [], []
        for j in range(S):
            commR_ref[0, j] = pq(bR0, j).astype(jnp.bfloat16)
            dR = mk(commR_ref, sendR_sems, recvR_sems, 0, j, right)
            dR.start()
            commL_ref[0, j] = pq(bL0, S + j).astype(jnp.bfloat16)
            dL = mk(commL_ref, sendL_sems, recvL_sems, 0, j, left)
            dL.start()
            curR.append(dR)
            curL.append(dL)
            all_rdmas.extend([dR, dL])

        pR = [pq(blkR(0), j).astype(jnp.bfloat16) for j in range(S)]
        pL = [pq(blkL(0), S + j).astype(jnp.bfloat16) for j in range(S)]

        for h in range(N_DEV - 1):
            last = h == N_DEV - 2
            nextR, nextL = [], []
            for j in range(S):
                curR[j].wait_recv()
                if not last:
                    commR_ref[h + 1, j] = commR_ref[h + 1, j] + pR[j]
                    dR = mk(commR_ref, sendR_sems, recvR_sems, h + 1, j, right)
                    dR.start()
                    nextR.append(dR)
                    all_rdmas.append(dR)
                else:
                    out_ref[:, j * swid:(j + 1) * swid] = (
                        commR_ref[h + 1, j].astype(jnp.float32) + pR[j]
                    )

                curL[j].wait_recv()
                if not last:
                    commL_ref[h + 1, j] = commL_ref[h + 1, j] + pL[j]
                    dL = mk(commL_ref, sendL_sems, recvL_sems, h + 1, j, left)
                    dL.start()
                    nextL.append(dL)
                    all_rdmas.append(dL)
                else:
                    out_ref[:, nh + j * swid:nh + (j + 1) * swid] = (
                        commL_ref[h + 1, j].astype(jnp.float32) + pL[j]
                    )
            curR, curL = nextR, nextL
            if h < N_DEV - 2:
                nlast = h + 1 == N_DEV - 2
                cast = (lambda v: v) if nlast else (
                    lambda v: v.astype(jnp.bfloat16))
                pR = [cast(pq(blkR(h + 1), j)) for j in range(S)]
                pL = [cast(pq(blkL(h + 1), S + j)) for j in range(S)]

        for d in all_rdmas:
            d.wait_send()

    return pl.pallas_call(
        body,
        out_shape=jax.ShapeDtypeStruct((M_BLK, n), jnp.float32),
        in_specs=[
            pl.BlockSpec(memory_space=pltpu.VMEM),
            pl.BlockSpec(memory_space=pltpu.VMEM),
        ],
        out_specs=pl.BlockSpec(memory_space=pltpu.VMEM),
        scratch_shapes=[
            pltpu.VMEM((m, k_per), jnp.bfloat16),
            pltpu.VMEM((N_DEV, S, M_BLK, swid), jnp.bfloat16),
            pltpu.VMEM((N_DEV, S, M_BLK, swid), jnp.bfloat16),
            pltpu.SemaphoreType.DMA((N_DEV - 1, S)),
            pltpu.SemaphoreType.DMA((N_DEV - 1, S)),
            pltpu.SemaphoreType.DMA((N_DEV - 1, S)),
            pltpu.SemaphoreType.DMA((N_DEV - 1, S)),
        ],
        compiler_params=pltpu.CompilerParams(collective_id=0),
    )(x, w_mat)
